# baseline (speedup 1.0000x reference)
import functools

import jax
import jax.numpy as jnp
from jax import lax
from jax.experimental import pallas as pl
from jax.experimental.pallas import tpu as pltpu
from jax.experimental.pallas import tpu_sc as plsc

N = 10000
E = 320000
D = 128

NC = 2
NS = 16
NW = NC * NS
CH = 128
CHUNKS_PAD = 2560
E_PAD = CHUNKS_PAD * CH
KPW1 = CHUNKS_PAD // NS
KPW2 = CHUNKS_PAD // NW
RPS = 632
NPAD = NS * RPS

_BM = 1000


def _mm3_body(x_ref, w_ref, b_ref, h0_ref, h1_ref, h2_ref):
    h = jnp.dot(x_ref[...], w_ref[...],
                preferred_element_type=jnp.float32,
                precision=lax.Precision.HIGHEST) + b_ref[...]
    h0_ref[...] = h[:, 0:D]
    h1_ref[...] = h[:, D:2 * D]
    h2_ref[...] = h[:, 2 * D:3 * D]


def _mm3(x, w, b):
    return pl.pallas_call(
        _mm3_body,
        grid=(N // _BM,),
        in_specs=[
            pl.BlockSpec((_BM, D), lambda i: (i, 0)),
            pl.BlockSpec((D, 3 * D), lambda i: (0, 0)),
            pl.BlockSpec((1, 3 * D), lambda i: (0, 0)),
        ],
        out_specs=[pl.BlockSpec((_BM, D), lambda i: (i, 0))] * 3,
        out_shape=[jax.ShapeDtypeStruct((N, D), jnp.float32)] * 3,
    )(x, w, b)


def _assemble_body(h0_ref, y1_ref, p0_ref, p1_ref, out_ref):
    out_ref[:, 0:D] = h0_ref[...]
    out_ref[:, D:2 * D] = y1_ref[...]
    out_ref[:, 2 * D:3 * D] = p0_ref[...] + p1_ref[...]


def _assemble(h0, y1, p0, p1):
    return pl.pallas_call(
        _assemble_body,
        grid=(N // _BM,),
        in_specs=[pl.BlockSpec((_BM, D), lambda i: (i, 0))] * 4,
        out_specs=pl.BlockSpec((_BM, 3 * D), lambda i: (i, 0)),
        out_shape=jax.ShapeDtypeStruct((N, 3 * D), jnp.float32),
    )(h0, y1, p0, p1)


_mesh = plsc.VectorSubcoreMesh(core_axis_name="c", subcore_axis_name="s",
                               num_cores=NC, num_subcores=NS)

_SC_OUT2 = (jax.ShapeDtypeStruct((NPAD, D), jnp.float32),
            jax.ShapeDtypeStruct((NPAD, D), jnp.float32))


SEG = 16


def _sc_scratch():
    return [
        pltpu.VMEM((SEG, CH), jnp.int32),
        pltpu.VMEM((SEG, CH), jnp.int32),
        pltpu.VMEM((CH, D), jnp.float32),
        pltpu.VMEM((CH, D), jnp.float32),
        pltpu.VMEM_SHARED((NPAD, D), jnp.float32),
        pltpu.SemaphoreType.DMA,
        pltpu.SemaphoreType.DMA,
    ]


def _zero_acc(r0, acc, s):
    @pl.loop(0, CH)
    def _(r):
        @pl.loop(0, D, step=16)
        def _(k):
            r0[r, pl.ds(k, 16)] = jnp.zeros((16,), jnp.float32)

    @pl.loop(0, 4)
    def _(j):
        pltpu.sync_copy(r0, acc.at[pl.ds(s * RPS + j * CH, CH)])

    pltpu.sync_copy(r0.at[pl.ds(0, 120)],
                    acc.at[pl.ds(s * RPS + 4 * CH, 120)])


def _run_edges(h_hbm, colp_hbm, rowp_hbm, idxc, idxr, r0, r1, acc,
               sem0, sem1, base, kpw):
    @pl.loop(0, kpw // SEG)
    def _(g):
        sbase = base + g * SEG
        pltpu.sync_copy(colp_hbm.at[pl.ds(sbase, SEG)], idxc)
        pltpu.sync_copy(rowp_hbm.at[pl.ds(sbase, SEG)], idxr)
        pltpu.async_copy(h_hbm.at[idxc.at[0]], r0, sem0)

        @pl.loop(0, SEG // 2)
        def _(t):
            i0 = 2 * t
            i1 = i0 + 1
            pltpu.make_async_copy(h_hbm.at[idxc.at[i0]], r0, sem0).wait()
            pltpu.async_copy(h_hbm.at[idxc.at[i1]], r1, sem1)
            pltpu.sync_copy(r0, acc.at[idxr.at[i0]], add=True)
            pltpu.make_async_copy(h_hbm.at[idxc.at[i1]], r1, sem1).wait()

            @pl.when(t < SEG // 2 - 1)
            def _():
                pltpu.async_copy(h_hbm.at[idxc.at[i0 + 2]], r0, sem0)

            pltpu.sync_copy(r1, acc.at[idxr.at[i1]], add=True)


@functools.partial(pl.kernel, out_type=_SC_OUT2, mesh=_mesh,
                   scratch_types=_sc_scratch())
def _spmm_pass1(h1_hbm, h2_hbm, colp_hbm, rowp_hbm, y1_hbm, y2_hbm,
                idxc, idxr, r0, r1, acc, sem0, sem1):
    c = lax.axis_index("c")
    s = lax.axis_index("s")
    _zero_acc(r0, acc, s)
    plsc.subcore_barrier()

    @pl.when(c == 0)
    def _():
        _run_edges(h1_hbm, colp_hbm, rowp_hbm, idxc, idxr, r0, r1, acc,
                   sem0, sem1, s * KPW1, KPW1)

    @pl.when(c == 1)
    def _():
        _run_edges(h2_hbm, colp_hbm, rowp_hbm, idxc, idxr, r0, r1, acc,
                   sem0, sem1, s * KPW1, KPW1)

    plsc.subcore_barrier()

    @pl.when(c == 0)
    def _():
        pltpu.sync_copy(acc.at[pl.ds(s * RPS, RPS)],
                        y1_hbm.at[pl.ds(s * RPS, RPS)])

    @pl.when(c == 1)
    def _():
        pltpu.sync_copy(acc.at[pl.ds(s * RPS, RPS)],
                        y2_hbm.at[pl.ds(s * RPS, RPS)])


@functools.partial(pl.kernel, out_type=_SC_OUT2, mesh=_mesh,
                   scratch_types=_sc_scratch())
def _spmm_pass2(h_hbm, colp_hbm, rowp_hbm, p0_hbm, p1_hbm,
                idxc, idxr, r0, r1, acc, sem0, sem1):
    c = lax.axis_index("c")
    s = lax.axis_index("s")
    _zero_acc(r0, acc, s)
    plsc.subcore_barrier()
    w = s * NC + c
    _run_edges(h_hbm, colp_hbm, rowp_hbm, idxc, idxr, r0, r1, acc,
               sem0, sem1, w * KPW2, KPW2)
    plsc.subcore_barrier()

    @pl.when(c == 0)
    def _():
        pltpu.sync_copy(acc.at[pl.ds(s * RPS, RPS)],
                        p0_hbm.at[pl.ds(s * RPS, RPS)])

    @pl.when(c == 1)
    def _():
        pltpu.sync_copy(acc.at[pl.ds(s * RPS, RPS)],
                        p1_hbm.at[pl.ds(s * RPS, RPS)])


def kernel(x, edge_index, W0, b0, W1, b1, W2, b2):
    ei = edge_index.astype(jnp.int32)
    row, col = ei[0], ei[1]
    pad = E_PAD - E
    colp = jnp.concatenate(
        [col, jnp.zeros((pad,), jnp.int32)]).reshape(CHUNKS_PAD, CH)
    rowp = jnp.concatenate(
        [row, N + (jnp.arange(pad, dtype=jnp.int32) % (NPAD - N))]
    ).reshape(CHUNKS_PAD, CH)
    w = jnp.concatenate([W0.T, W1.T, W2.T], axis=1)
    b = jnp.concatenate([b0, b1, b2]).reshape(1, 3 * D)
    h0, h1, h2 = _mm3(x, w, b)
    y1, y2a = _spmm_pass1(h1, h2, colp, rowp)
    p0, p1 = _spmm_pass2(y2a, colp, rowp)
    return _assemble(h0, y1, p0, p1)
